# baseline (speedup 1.0000x reference)
import math

import jax
import jax.numpy as jnp
from jax.experimental import pallas as pl
from jax.experimental.pallas import tpu as pltpu

_BB = 16


def _body(onehot_ref, mem_hbm, emb_ref, qWT_ref, qb_ref, oWT_ref, ob_ref,
          out_ref, buf, sems):
    D = emb_ref.shape[1]
    B = mem_hbm.shape[0]
    nsteps = B // _BB
    i = pl.program_id(0)

    hb = _BB // 2

    def dmas(block_idx, slot):
        d0 = pltpu.make_async_copy(
            mem_hbm.at[pl.ds(block_idx * _BB, hb)],
            buf.at[slot, 0:hb], sems.at[slot])
        d1 = pltpu.make_async_copy(
            mem_hbm.at[pl.ds(block_idx * _BB + hb, hb)],
            buf.at[slot, hb:_BB], sems.at[slot])
        return d0, d1

    def start(block_idx, slot):
        d0, d1 = dmas(block_idx, slot)
        d0.start(priority=0)
        d1.start(priority=1)

    @pl.when(i == 0)
    def _():
        start(0, 0)

    @pl.when(i + 1 < nsteps)
    def _():
        start(i + 1, (i + 1) % 2)

    d0, d1 = dmas(i, i % 2)
    d0.wait()
    d1.wait()
    mem = buf.at[i % 2]

    e = jnp.dot(onehot_ref[...], emb_ref[...],
                preferred_element_type=jnp.float32)
    q = jnp.dot(e, qWT_ref[...], preferred_element_type=jnp.float32)
    q = (q + qb_ref[...]) * (1.0 / math.sqrt(D))

    S_list = []
    for b in range(_BB):
        S = jax.lax.dot_general(
            q[b:b + 1], mem[b],
            dimension_numbers=(((1,), (1,)), ((), ())),
            preferred_element_type=jnp.float32)
        S_list.append(S)

    p_list = []
    s_list = []
    for b in range(_BB):
        S = S_list[b]
        mx = jnp.max(S, axis=1, keepdims=True)
        p = jnp.exp(S - mx)
        s_list.append(jnp.sum(p, axis=1, keepdims=True))
        p_list.append(p)

    pooled_rows = []
    for b in range(_BB):
        P = jax.lax.dot_general(
            p_list[b], mem[b],
            dimension_numbers=(((1,), (0,)), ((), ())),
            preferred_element_type=jnp.float32)
        pooled_rows.append(P)

    pooled = jnp.concatenate(pooled_rows, axis=0)
    SS = jnp.concatenate(s_list, axis=0)

    acc = jnp.dot(pooled, oWT_ref[...], preferred_element_type=jnp.float32)
    out_ref[...] = acc * (1.0 / SS) + ob_ref[...]


def kernel(query, memory, emb, qW, qb, oW, ob):
    B, M, D = memory.shape
    V = oW.shape[0]
    onehot = jax.nn.one_hot(query, emb.shape[0], dtype=jnp.float32)
    qWT = qW.T
    oWT = oW.T
    qb2 = qb.reshape(1, D)
    ob2 = ob.reshape(1, V)

    grid = (B // _BB,)
    out = pl.pallas_call(
        _body,
        out_shape=jax.ShapeDtypeStruct((B, V), jnp.float32),
        grid=grid,
        in_specs=[
            pl.BlockSpec((_BB, emb.shape[0]), lambda i: (i, 0)),
            pl.BlockSpec(memory_space=pl.ANY),
            pl.BlockSpec(emb.shape, lambda i: (0, 0)),
            pl.BlockSpec((D, D), lambda i: (0, 0)),
            pl.BlockSpec((1, D), lambda i: (0, 0)),
            pl.BlockSpec((D, V), lambda i: (0, 0)),
            pl.BlockSpec((1, V), lambda i: (0, 0)),
        ],
        out_specs=pl.BlockSpec((_BB, V), lambda i: (i, 0)),
        scratch_shapes=[
            pltpu.VMEM((2, _BB, M, D), jnp.float32),
            pltpu.SemaphoreType.DMA((2,)),
        ],
        compiler_params=pltpu.CompilerParams(
            dimension_semantics=("arbitrary",),
            vmem_limit_bytes=48 * 1024 * 1024,
        ),
        name="read_gate_fused",
    )(onehot, memory, emb, qWT, qb2, oWT, ob2)
    return out


# speedup vs baseline: 1.0423x; 1.0423x over previous
import math

import jax
import jax.numpy as jnp
from jax.experimental import pallas as pl
from jax.experimental.pallas import tpu as pltpu

_BB = 8


def _body(onehot_ref, mem_hbm, emb_ref, qWT_ref, qb_ref, oWT_ref, ob_ref,
          out_ref, buf, sems):
    D = emb_ref.shape[1]
    B = mem_hbm.shape[0]
    nsteps = B // _BB
    i = pl.program_id(0)

    hb = _BB // 2

    def dmas(block_idx, slot):
        d0 = pltpu.make_async_copy(
            mem_hbm.at[pl.ds(block_idx * _BB, hb)],
            buf.at[slot, 0:hb], sems.at[slot])
        d1 = pltpu.make_async_copy(
            mem_hbm.at[pl.ds(block_idx * _BB + hb, hb)],
            buf.at[slot, hb:_BB], sems.at[slot])
        return d0, d1

    def start(block_idx, slot):
        d0, d1 = dmas(block_idx, slot)
        d0.start(priority=0)
        d1.start(priority=1)

    @pl.when(i == 0)
    def _():
        start(0, 0)

    @pl.when(i + 1 < nsteps)
    def _():
        start(i + 1, (i + 1) % 2)

    d0, d1 = dmas(i, i % 2)
    d0.wait()
    d1.wait()
    mem = buf.at[i % 2]

    e = jnp.dot(onehot_ref[...], emb_ref[...],
                preferred_element_type=jnp.float32)
    q = jnp.dot(e, qWT_ref[...], preferred_element_type=jnp.float32)
    q = (q + qb_ref[...]) * (1.0 / math.sqrt(D))

    S_list = []
    for b in range(_BB):
        S = jax.lax.dot_general(
            q[b:b + 1], mem[b],
            dimension_numbers=(((1,), (1,)), ((), ())),
            preferred_element_type=jnp.float32)
        S_list.append(S)

    p_list = []
    s_list = []
    for b in range(_BB):
        S = S_list[b]
        mx = jnp.max(S, axis=1, keepdims=True)
        p = jnp.exp(S - mx)
        s_list.append(jnp.sum(p, axis=1, keepdims=True))
        p_list.append(p)

    pooled_rows = []
    for b in range(_BB):
        P = jax.lax.dot_general(
            p_list[b], mem[b],
            dimension_numbers=(((1,), (0,)), ((), ())),
            preferred_element_type=jnp.float32)
        pooled_rows.append(P)

    pooled = jnp.concatenate(pooled_rows, axis=0)
    SS = jnp.concatenate(s_list, axis=0)

    acc = jnp.dot(pooled, oWT_ref[...], preferred_element_type=jnp.float32)
    out_ref[...] = acc * (1.0 / SS) + ob_ref[...]


def kernel(query, memory, emb, qW, qb, oW, ob):
    B, M, D = memory.shape
    V = oW.shape[0]
    onehot = jax.nn.one_hot(query, emb.shape[0], dtype=jnp.float32)
    qWT = qW.T
    oWT = oW.T
    qb2 = qb.reshape(1, D)
    ob2 = ob.reshape(1, V)

    grid = (B // _BB,)
    out = pl.pallas_call(
        _body,
        out_shape=jax.ShapeDtypeStruct((B, V), jnp.float32),
        grid=grid,
        in_specs=[
            pl.BlockSpec((_BB, emb.shape[0]), lambda i: (i, 0)),
            pl.BlockSpec(memory_space=pl.ANY),
            pl.BlockSpec(emb.shape, lambda i: (0, 0)),
            pl.BlockSpec((D, D), lambda i: (0, 0)),
            pl.BlockSpec((1, D), lambda i: (0, 0)),
            pl.BlockSpec((D, V), lambda i: (0, 0)),
            pl.BlockSpec((1, V), lambda i: (0, 0)),
        ],
        out_specs=pl.BlockSpec((_BB, V), lambda i: (i, 0)),
        scratch_shapes=[
            pltpu.VMEM((2, _BB, M, D), jnp.float32),
            pltpu.SemaphoreType.DMA((2,)),
        ],
        compiler_params=pltpu.CompilerParams(
            dimension_semantics=("arbitrary",),
            vmem_limit_bytes=48 * 1024 * 1024,
        ),
        name="read_gate_fused",
    )(onehot, memory, emb, qWT, qb2, oWT, ob2)
    return out


# speedup vs baseline: 1.0438x; 1.0015x over previous
import math

import jax
import jax.numpy as jnp
from jax.experimental import pallas as pl
from jax.experimental.pallas import tpu as pltpu

_BB = 8


def _body(onehot_ref, mem_hbm, emb_ref, qWT_ref, qb_ref, oWT_ref, ob_ref,
          out_ref, buf, sems):
    D = emb_ref.shape[1]
    B = mem_hbm.shape[0]
    nsteps = B // _BB
    i = pl.program_id(0)

    M = mem_hbm.shape[1]
    nt = _BB * M // 8
    mem_t = mem_hbm.reshape(B * M // 8, 8, D)

    def dma(block_idx, slot):
        return pltpu.make_async_copy(
            mem_t.at[pl.ds(block_idx * nt, nt)], buf.at[slot],
            sems.at[slot])

    @pl.when(i == 0)
    def _():
        dma(0, 0).start()

    @pl.when(i + 1 < nsteps)
    def _():
        dma(i + 1, (i + 1) % 2).start()

    dma(i, i % 2).wait()
    mem = buf.at[i % 2].reshape(_BB * M, D)

    e = jnp.dot(onehot_ref[...], emb_ref[...],
                preferred_element_type=jnp.float32)
    q = jnp.dot(e, qWT_ref[...], preferred_element_type=jnp.float32)
    q = (q + qb_ref[...]) * (1.0 / math.sqrt(D))

    S_list = []
    for b in range(_BB):
        S = jax.lax.dot_general(
            q[b:b + 1], mem[pl.ds(b * (mem.shape[0] // _BB), mem.shape[0] // _BB)],
            dimension_numbers=(((1,), (1,)), ((), ())),
            preferred_element_type=jnp.float32)
        S_list.append(S)

    p_list = []
    s_list = []
    for b in range(_BB):
        S = S_list[b]
        mx = jnp.max(S, axis=1, keepdims=True)
        p = jnp.exp(S - mx)
        s_list.append(jnp.sum(p, axis=1, keepdims=True))
        p_list.append(p)

    pooled_rows = []
    for b in range(_BB):
        P = jax.lax.dot_general(
            p_list[b], mem[pl.ds(b * (mem.shape[0] // _BB), mem.shape[0] // _BB)],
            dimension_numbers=(((1,), (0,)), ((), ())),
            preferred_element_type=jnp.float32)
        pooled_rows.append(P)

    pooled = jnp.concatenate(pooled_rows, axis=0)
    SS = jnp.concatenate(s_list, axis=0)

    acc = jnp.dot(pooled, oWT_ref[...], preferred_element_type=jnp.float32)
    out_ref[...] = acc * (1.0 / SS) + ob_ref[...]


def kernel(query, memory, emb, qW, qb, oW, ob):
    B, M, D = memory.shape
    V = oW.shape[0]
    onehot = jax.nn.one_hot(query, emb.shape[0], dtype=jnp.float32)
    qWT = qW.T
    oWT = oW.T
    qb2 = qb.reshape(1, D)
    ob2 = ob.reshape(1, V)

    grid = (B // _BB,)
    out = pl.pallas_call(
        _body,
        out_shape=jax.ShapeDtypeStruct((B, V), jnp.float32),
        grid=grid,
        in_specs=[
            pl.BlockSpec((_BB, emb.shape[0]), lambda i: (i, 0)),
            pl.BlockSpec(memory_space=pl.ANY),
            pl.BlockSpec(emb.shape, lambda i: (0, 0)),
            pl.BlockSpec((D, D), lambda i: (0, 0)),
            pl.BlockSpec((1, D), lambda i: (0, 0)),
            pl.BlockSpec((D, V), lambda i: (0, 0)),
            pl.BlockSpec((1, V), lambda i: (0, 0)),
        ],
        out_specs=pl.BlockSpec((_BB, V), lambda i: (i, 0)),
        scratch_shapes=[
            pltpu.VMEM((2, _BB * M // 8, 8, D), jnp.float32),
            pltpu.SemaphoreType.DMA((2,)),
        ],
        compiler_params=pltpu.CompilerParams(
            dimension_semantics=("arbitrary",),
            vmem_limit_bytes=48 * 1024 * 1024,
        ),
        name="read_gate_fused",
    )(onehot, memory, emb, qWT, qb2, oWT, ob2)
    return out
